# Initial kernel scaffold; baseline (speedup 1.0000x reference)
import jax
import jax.numpy as jnp
from jax.experimental import pallas as pl


def kernel(x, edge_index, batch, W0, att_src0, att_dst0, b0, W1, att_src1, att_dst1, b1, W2, att_src2, att_dst2, b2, proj_W, proj_b):
    raise NotImplementedError("write your pallas kernel here")


# speedup vs baseline: 16.6483x; 16.6483x over previous
import functools

import jax
import jax.numpy as jnp
from jax import lax
from jax.experimental import pallas as pl
from jax.experimental.pallas import tpu as pltpu
from jax.experimental.pallas import tpu_sc as plsc

N = 10000
E = 320000
IN_DIM = 128
HID = 64
HEADS = 4
NUM_GRAPHS = 128

NP = 10240
RANGE = 320
ETOT = E + N
ECH = 2048
NCH = (ETOT + ECH - 1) // ECH
EP = NCH * ECH
CAP = EP + ECH
ACCW = (RANGE + 1) * 256

_mesh = plsc.VectorSubcoreMesh(core_axis_name="c", subcore_axis_name="s")


def _wid():
    return lax.axis_index("s") * 2 + lax.axis_index("c")


def _s0_body(src_hbm, dst_hbm, psrc, pdst, cnts, sbuf_s, sbuf_d, sel_s, sel_d, cbuf):
    wid = _wid()
    d0 = wid * RANGE

    def chunk(c, carry):
        staged, flushed = carry
        pltpu.sync_copy(src_hbm.at[pl.ds(c * ECH, ECH)], sbuf_s)
        pltpu.sync_copy(dst_hbm.at[pl.ds(c * ECH, ECH)], sbuf_d)

        def vloop(v, st):
            sv = sbuf_s[pl.ds(v * 16, 16)]
            dv = sbuf_d[pl.ds(v * 16, 16)]
            d0v = jnp.full((16,), d0, jnp.int32)
            m = (dv >= d0v) & (dv < d0v + RANGE)
            mi = jnp.where(m, jnp.full((16,), 1, jnp.int32), jnp.full((16,), 0, jnp.int32))
            cs = plsc.cumsum(mi)
            pos = jnp.full((16,), st, jnp.int32) + cs - mi
            plsc.store_scatter(sel_s, [pos], sv, mask=m)
            plsc.store_scatter(sel_d, [pos], dv - d0v, mask=m)
            return st + jnp.max(plsc.all_reduce_population_count(m))

        staged = lax.fori_loop(0, ECH // 16, vloop, staged)
        pred = staged >= ECH

        @pl.when(pred)
        def _():
            pltpu.sync_copy(sel_s.at[pl.ds(0, ECH)], psrc.at[pl.ds(pl.multiple_of(wid * CAP + flushed, 8), ECH)])
            pltpu.sync_copy(sel_d.at[pl.ds(0, ECH)], pdst.at[pl.ds(pl.multiple_of(wid * CAP + flushed, 8), ECH)])

            def mv(v, _):
                sel_s[pl.ds(v * 16, 16)] = sel_s[pl.ds(ECH + v * 16, 16)]
                sel_d[pl.ds(v * 16, 16)] = sel_d[pl.ds(ECH + v * 16, 16)]
                return 0

            lax.fori_loop(0, ECH // 16, mv, 0)

        staged = jnp.where(pred, staged - ECH, staged)
        flushed = jnp.where(pred, flushed + ECH, flushed)
        return staged, flushed

    staged, flushed = lax.fori_loop(0, NCH, chunk, (jnp.int32(0), jnp.int32(0)))
    pltpu.sync_copy(sel_s.at[pl.ds(0, ECH)], psrc.at[pl.ds(pl.multiple_of(wid * CAP + flushed, 8), ECH)])
    pltpu.sync_copy(sel_d.at[pl.ds(0, ECH)], pdst.at[pl.ds(pl.multiple_of(wid * CAP + flushed, 8), ECH)])
    cbuf[...] = jnp.full((16,), flushed + staged, jnp.int32)
    pltpu.sync_copy(cbuf, cnts.at[pl.ds(wid * 16, 16)])


@jax.jit
def _s0(src, dst):
    f = pl.kernel(
        _s0_body,
        mesh=_mesh,
        compiler_params=pltpu.CompilerParams(needs_layout_passes=False),
        out_type=(
            jax.ShapeDtypeStruct((32 * CAP,), jnp.int32),
            jax.ShapeDtypeStruct((32 * CAP,), jnp.int32),
            jax.ShapeDtypeStruct((32 * 16,), jnp.int32),
        ),
        scratch_types=[
            pltpu.VMEM((ECH,), jnp.int32),
            pltpu.VMEM((ECH,), jnp.int32),
            pltpu.VMEM((2 * ECH + 16,), jnp.int32),
            pltpu.VMEM((2 * ECH + 16,), jnp.int32),
            pltpu.VMEM((16,), jnp.int32),
        ],
    )
    return f(src, dst)


def _s1_body(psrc, pdst, cnts, ast_hbm, adt_hbm, mout, sout, aout,
             tab_s, tab_d, macc, sacc, sbuf_s, sbuf_d, abuf, obm, obs, cbuf):
    wid = _wid()
    d0 = wid * RANGE
    iv = lax.iota(jnp.int32, 16)
    hv = iv & 3
    i4 = iv >> 2

    pltpu.sync_copy(ast_hbm, tab_s)
    pltpu.sync_copy(adt_hbm, tab_d)
    pltpu.sync_copy(cnts.at[pl.ds(wid * 16, 16)], cbuf)
    cnt = cbuf[pl.ds(0, 16)][0]

    def init(i, _):
        macc[pl.ds(i * 16, 16)] = jnp.full((16,), -3e38, jnp.float32)
        sacc[pl.ds(i * 16, 16)] = jnp.zeros((16,), jnp.float32)
        return 0

    lax.fori_loop(0, 4 * RANGE * 4 // 16, init, 0)

    nch = (cnt + ECH - 1) // ECH

    def chunk(c, _):
        pltpu.sync_copy(psrc.at[pl.ds(pl.multiple_of(wid * CAP + c * ECH, 8), ECH)], sbuf_s)
        pltpu.sync_copy(pdst.at[pl.ds(pl.multiple_of(wid * CAP + c * ECH, 8), ECH)], sbuf_d)

        def v(vi, _):
            eids = jnp.full((16,), vi * 4, jnp.int32) + i4
            valid = eids < jnp.full((16,), cnt - c * ECH, jnp.int32)
            srcx = plsc.load_gather(sbuf_s, [eids])
            dlx = plsc.load_gather(sbuf_d, [eids])
            d0v = jnp.full((16,), d0, jnp.int32)
            a_s = plsc.load_gather(tab_s, [srcx * 4 + hv], mask=valid)
            a_d = plsc.load_gather(tab_d, [(dlx + d0v) * 4 + hv], mask=valid)
            al = a_s + a_d
            al = jnp.where(al > 0, al, 0.2 * al)
            abuf[pl.ds(vi * 16, 16)] = al
            iacc = i4 * (RANGE * 4) + dlx * 4 + hv
            mo = plsc.load_gather(macc, [iacc], mask=valid)
            so = plsc.load_gather(sacc, [iacc], mask=valid)
            mn = jnp.maximum(mo, al)
            sn = so * jnp.exp(mo - mn) + jnp.exp(al - mn)
            plsc.store_scatter(macc, [iacc], mn, mask=valid)
            plsc.store_scatter(sacc, [iacc], sn, mask=valid)
            return 0

        lax.fori_loop(0, ECH // 4, v, 0)
        pltpu.sync_copy(abuf, aout.at[pl.ds(pl.multiple_of((wid * CAP + c * ECH) * 4, 8), ECH * 4)])
        return 0

    lax.fori_loop(0, nch, chunk, 0)

    def comb(j, _):
        off = pl.ds(j * 16, 16)
        m0 = macc[pl.ds(0 * RANGE * 4 + j * 16, 16)]
        m1 = macc[pl.ds(1 * RANGE * 4 + j * 16, 16)]
        m2 = macc[pl.ds(2 * RANGE * 4 + j * 16, 16)]
        m3 = macc[pl.ds(3 * RANGE * 4 + j * 16, 16)]
        s0 = sacc[pl.ds(0 * RANGE * 4 + j * 16, 16)]
        s1 = sacc[pl.ds(1 * RANGE * 4 + j * 16, 16)]
        s2 = sacc[pl.ds(2 * RANGE * 4 + j * 16, 16)]
        s3 = sacc[pl.ds(3 * RANGE * 4 + j * 16, 16)]
        M = jnp.maximum(jnp.maximum(m0, m1), jnp.maximum(m2, m3))
        S = (s0 * jnp.exp(m0 - M) + s1 * jnp.exp(m1 - M)
             + s2 * jnp.exp(m2 - M) + s3 * jnp.exp(m3 - M))
        obm[off] = M
        obs[off] = S
        return 0

    lax.fori_loop(0, RANGE * 4 // 16, comb, 0)
    pltpu.sync_copy(obm, mout.at[pl.ds(wid * RANGE * 4, RANGE * 4)])
    pltpu.sync_copy(obs, sout.at[pl.ds(wid * RANGE * 4, RANGE * 4)])


@jax.jit
def _s1(psrc, pdst, cnts, ast, adt):
    f = pl.kernel(
        _s1_body,
        mesh=_mesh,
        compiler_params=pltpu.CompilerParams(needs_layout_passes=False),
        out_type=(
            jax.ShapeDtypeStruct((32 * RANGE * 4,), jnp.float32),
            jax.ShapeDtypeStruct((32 * RANGE * 4,), jnp.float32),
            jax.ShapeDtypeStruct((32 * CAP * 4,), jnp.float32),
        ),
        scratch_types=[
            pltpu.VMEM((NP * 4,), jnp.float32),
            pltpu.VMEM((NP * 4,), jnp.float32),
            pltpu.VMEM((4 * RANGE * 4,), jnp.float32),
            pltpu.VMEM((4 * RANGE * 4,), jnp.float32),
            pltpu.VMEM((ECH,), jnp.int32),
            pltpu.VMEM((ECH,), jnp.int32),
            pltpu.VMEM((ECH * 4,), jnp.float32),
            pltpu.VMEM((RANGE * 4,), jnp.float32),
            pltpu.VMEM((RANGE * 4,), jnp.float32),
            pltpu.VMEM((16,), jnp.int32),
        ],
    )
    return f(psrc, pdst, cnts, ast, adt)


def _s2_body(psrc, pdst, cnts, aval, mrow, srow, h_hbm, ohbm,
             acc, sbuf_s, sbuf_d, abuf, mb, sb, rows, cbuf, sem):
    wid = _wid()
    iv = lax.iota(jnp.int32, 16)
    hv = iv & 3
    i4 = iv >> 2

    pltpu.sync_copy(cnts.at[pl.ds(wid * 16, 16)], cbuf)
    cnt = cbuf[pl.ds(0, 16)][0]
    pltpu.sync_copy(mrow.at[pl.ds(wid * RANGE * 4, RANGE * 4)], mb.at[pl.ds(0, RANGE * 4)])
    pltpu.sync_copy(srow.at[pl.ds(wid * RANGE * 4, RANGE * 4)], sb.at[pl.ds(0, RANGE * 4)])

    def init(i, _):
        acc[pl.ds(i * 16, 16)] = jnp.zeros((16,), jnp.float32)
        return 0

    lax.fori_loop(0, ACCW // 16, init, 0)

    nch = (cnt + ECH - 1) // ECH

    def chunk(c, _):
        pltpu.sync_copy(psrc.at[pl.ds(pl.multiple_of(wid * CAP + c * ECH, 8), ECH)], sbuf_s)
        pltpu.sync_copy(pdst.at[pl.ds(pl.multiple_of(wid * CAP + c * ECH, 8), ECH)], sbuf_d.at[pl.ds(0, ECH)])
        pltpu.sync_copy(aval.at[pl.ds(pl.multiple_of((wid * CAP + c * ECH) * 4, 8), ECH * 4)], abuf.at[pl.ds(0, ECH * 4)])
        rem = jnp.minimum(cnt - c * ECH, ECH)

        def pre(vi, _):
            off = pl.ds(vi * 16, 16)
            sv = sbuf_s[off]
            dv = sbuf_d[off]
            valid = (jnp.full((16,), vi * 16, jnp.int32) + iv) < jnp.full((16,), rem, jnp.int32)
            sbuf_s[off] = jnp.where(valid, sv, 0)
            sbuf_d[off] = jnp.where(valid, dv, RANGE)
            return 0

        lax.fori_loop(0, ECH // 16, pre, 0)

        def pre2(vi, _):
            off = pl.ds(vi * 16, 16)
            eids = jnp.full((16,), vi * 4, jnp.int32) + i4
            dlx = plsc.load_gather(sbuf_d, [eids])
            alv = abuf[off]
            mg = plsc.load_gather(mb, [dlx * 4 + hv])
            sg = plsc.load_gather(sb, [dlx * 4 + hv])
            abuf[off] = jnp.exp(alv - mg) / (sg + 1e-16)
            return 0

        lax.fori_loop(0, ECH // 4, pre2, 0)

        nb = (rem + 15) // 16

        def batch(b, _):
            pltpu.async_copy(h_hbm.at[sbuf_s.at[pl.ds(b * 16, 16)]], rows, sem).wait()

            dvec = sbuf_d[pl.ds(b * 16, 16)]
            avecs = [abuf[pl.ds(b * 64 + j * 16, 16)] for j in range(4)]
            for i in range(16):
                dl = dvec[i]
                base = dl * 256
                av = avecs[i // 4]
                for f in range(16):
                    asc = jnp.full((16,), av[(i % 4) * 4 + f // 4], jnp.float32)
                    plsc.addupdate(acc.at[pl.ds(base + f * 16, 16)],
                                   rows[i, pl.ds(f * 16, 16)] * asc)
            return 0

        lax.fori_loop(0, nb, batch, 0)
        return 0

    lax.fori_loop(0, nch, chunk, 0)
    pltpu.sync_copy(acc.at[pl.ds(0, RANGE * 256)], ohbm.at[pl.ds(wid * RANGE * 256, RANGE * 256)])


@jax.jit
def _s2(psrc, pdst, cnts, aval, mrow, srow, h):
    f = pl.kernel(
        _s2_body,
        mesh=_mesh,
        compiler_params=pltpu.CompilerParams(needs_layout_passes=False),
        out_type=jax.ShapeDtypeStruct((32 * RANGE * 256,), jnp.float32),
        scratch_types=[
            pltpu.VMEM((ACCW,), jnp.float32),
            pltpu.VMEM((ECH,), jnp.int32),
            pltpu.VMEM((ECH + 16,), jnp.int32),
            pltpu.VMEM((ECH * 4 + 16,), jnp.float32),
            pltpu.VMEM((RANGE * 4 + 16,), jnp.float32),
            pltpu.VMEM((RANGE * 4 + 16,), jnp.float32),
            pltpu.VMEM((16, 256), jnp.float32),
            pltpu.VMEM((16,), jnp.int32),
            pltpu.SemaphoreType.DMA,
        ],
    )
    return f(psrc, pdst, cnts, aval, mrow, srow, h)


def _t1_body(x_ref, w_ref, as_ref, ad_ref, h_ref, asr_ref, adr_ref):
    h = jnp.dot(x_ref[...], w_ref[...], preferred_element_type=jnp.float32)
    h_ref[...] = h
    asr_ref[...] = jnp.dot(h, as_ref[...], preferred_element_type=jnp.float32)
    adr_ref[...] = jnp.dot(h, ad_ref[...], preferred_element_type=jnp.float32)


def _t1(x, w, As, Ad):
    d = x.shape[1]
    return pl.pallas_call(
        _t1_body,
        grid=(8,),
        in_specs=[
            pl.BlockSpec((NP // 8, d), lambda i: (i, 0)),
            pl.BlockSpec((d, 256), lambda i: (0, 0)),
            pl.BlockSpec((256, 4), lambda i: (0, 0)),
            pl.BlockSpec((256, 4), lambda i: (0, 0)),
        ],
        out_specs=[
            pl.BlockSpec((NP // 8, 256), lambda i: (i, 0)),
            pl.BlockSpec((NP // 8, 4), lambda i: (i, 0)),
            pl.BlockSpec((NP // 8, 4), lambda i: (i, 0)),
        ],
        out_shape=[
            jax.ShapeDtypeStruct((NP, 256), jnp.float32),
            jax.ShapeDtypeStruct((NP, 4), jnp.float32),
            jax.ShapeDtypeStruct((NP, 4), jnp.float32),
        ],
    )(x, w, As, Ad)


def _t2_body(o_ref, b_ref, batch_ref, hn_ref, pool_ref):
    o = o_ref[...]
    mean = (o[:, 0:64] + o[:, 64:128] + o[:, 128:192] + o[:, 192:256]) * 0.25
    hn = jnp.maximum(mean + b_ref[...], 0.0)
    hn_ref[...] = hn
    bv = batch_ref[0, 0, :]
    gid = lax.broadcasted_iota(jnp.int32, (NP // 8, NUM_GRAPHS), 1).astype(jnp.float32)
    oh = (bv[:, None] == gid).astype(jnp.float32)
    pp = lax.dot_general(oh, hn, (((0,), (0,)), ((), ())),
                         preferred_element_type=jnp.float32)

    @pl.when(pl.program_id(0) == 0)
    def _():
        pool_ref[...] = jnp.zeros_like(pool_ref)

    pool_ref[...] += pp


def _t2(o, b, batch3):
    return pl.pallas_call(
        _t2_body,
        grid=(8,),
        in_specs=[
            pl.BlockSpec((NP // 8, 256), lambda i: (i, 0)),
            pl.BlockSpec((1, HID), lambda i: (0, 0)),
            pl.BlockSpec((1, 1, NP // 8), lambda i: (i, 0, 0)),
        ],
        out_specs=[
            pl.BlockSpec((NP // 8, HID), lambda i: (i, 0)),
            pl.BlockSpec((NUM_GRAPHS, HID), lambda i: (0, 0)),
        ],
        out_shape=[
            jax.ShapeDtypeStruct((NP, HID), jnp.float32),
            jax.ShapeDtypeStruct((NUM_GRAPHS, HID), jnp.float32),
        ],
    )(o, b, batch3)


def _t3_body(p0_ref, p1_ref, p2_ref, pw_ref, pb_ref, out_ref):
    p0, p1, p2 = p0_ref[...], p1_ref[...], p2_ref[...]
    pw = pw_ref[...]
    pb = pb_ref[0, 0]
    s0 = jnp.dot(p0, pw, preferred_element_type=jnp.float32) + pb
    s1 = jnp.dot(p1, pw, preferred_element_type=jnp.float32) + pb
    s2 = jnp.dot(p2, pw, preferred_element_type=jnp.float32) + pb
    m = jnp.maximum(jnp.maximum(s0, s1), s2)
    e0 = jnp.exp(s0 - m)
    e1 = jnp.exp(s1 - m)
    e2 = jnp.exp(s2 - m)
    out_ref[...] = (e0 * p0 + e1 * p1 + e2 * p2) / (e0 + e1 + e2)


def _t3(p0, p1, p2, pw, pb):
    return pl.pallas_call(
        _t3_body,
        in_specs=[
            pl.BlockSpec((NUM_GRAPHS, HID), lambda: (0, 0)),
            pl.BlockSpec((NUM_GRAPHS, HID), lambda: (0, 0)),
            pl.BlockSpec((NUM_GRAPHS, HID), lambda: (0, 0)),
            pl.BlockSpec((HID, 1), lambda: (0, 0)),
            pl.BlockSpec((1, 1), lambda: (0, 0)),
        ],
        out_specs=pl.BlockSpec((NUM_GRAPHS, HID), lambda: (0, 0)),
        out_shape=jax.ShapeDtypeStruct((NUM_GRAPHS, HID), jnp.float32),
    )(p0, p1, p2, pw, pb)


def _attmat(att):
    return (att[0][:, :, None] * jnp.eye(HEADS, dtype=jnp.float32)[:, None, :]).reshape(HEADS * HID, HEADS)


def kernel(x, edge_index, batch, W0, att_src0, att_dst0, b0, W1, att_src1, att_dst1, b1, W2, att_src2, att_dst2, b2, proj_W, proj_b):
    loop = jnp.arange(N, dtype=jnp.int32)
    src = jnp.concatenate([edge_index[0], loop,
                           jnp.zeros((EP - ETOT,), jnp.int32)])
    dst = jnp.concatenate([edge_index[1], loop,
                           jnp.full((EP - ETOT,), NP - 1, jnp.int32)])
    psrc, pdst, cnts = _s0(src, dst)

    x_pad = jnp.concatenate([x, jnp.zeros((NP - N, IN_DIM), jnp.float32)])
    batch3 = jnp.concatenate([batch, jnp.full((NP - N,), NUM_GRAPHS, jnp.int32)]
                             ).astype(jnp.float32).reshape(8, 1, NP // 8)

    params = [(W0, att_src0, att_dst0, b0), (W1, att_src1, att_dst1, b1),
              (W2, att_src2, att_dst2, b2)]
    h = x_pad
    pooled = []
    for (W, a_s, a_d, b) in params:
        h256, asr, adr = _t1(h, W, _attmat(a_s), _attmat(a_d))
        m32, s32, aval = _s1(psrc, pdst, cnts, asr.reshape(-1), adr.reshape(-1))
        o32 = _s2(psrc, pdst, cnts, aval, m32, s32, h256)
        h, pool = _t2(o32.reshape(NP, 256), b.reshape(1, HID), batch3)
        pooled.append(pool)

    return _t3(pooled[0], pooled[1], pooled[2], proj_W, proj_b.reshape(1, 1))


# speedup vs baseline: 18.0162x; 1.0822x over previous
import functools

import jax
import jax.numpy as jnp
from jax import lax
from jax.experimental import pallas as pl
from jax.experimental.pallas import tpu as pltpu
from jax.experimental.pallas import tpu_sc as plsc

N = 10000
E = 320000
IN_DIM = 128
HID = 64
HEADS = 4
NUM_GRAPHS = 128

NP = 10240
RANGE = 320
ETOT = E + N
ECH = 2048
NCH = (ETOT + ECH - 1) // ECH
EP = NCH * ECH
CAP = EP + ECH
ACCW = (RANGE + 1) * 256

_mesh = plsc.VectorSubcoreMesh(core_axis_name="c", subcore_axis_name="s")


def _wid():
    return lax.axis_index("s") * 2 + lax.axis_index("c")


def _s0_body(src_hbm, dst_hbm, psrc, pdst, cnts, sbuf_s, sbuf_d, sel_s, sel_d, cbuf):
    wid = _wid()
    d0 = wid * RANGE

    def chunk(c, carry):
        staged, flushed = carry
        pltpu.sync_copy(src_hbm.at[pl.ds(c * ECH, ECH)], sbuf_s)
        pltpu.sync_copy(dst_hbm.at[pl.ds(c * ECH, ECH)], sbuf_d)

        def vloop(v, st):
            sv = sbuf_s[pl.ds(v * 16, 16)]
            dv = sbuf_d[pl.ds(v * 16, 16)]
            d0v = jnp.full((16,), d0, jnp.int32)
            m = (dv >= d0v) & (dv < d0v + RANGE)
            mi = jnp.where(m, jnp.full((16,), 1, jnp.int32), jnp.full((16,), 0, jnp.int32))
            cs = plsc.cumsum(mi)
            pos = jnp.full((16,), st, jnp.int32) + cs - mi
            plsc.store_scatter(sel_s, [pos], sv, mask=m)
            plsc.store_scatter(sel_d, [pos], dv - d0v, mask=m)
            return st + jnp.max(plsc.all_reduce_population_count(m))

        staged = lax.fori_loop(0, ECH // 16, vloop, staged)
        pred = staged >= ECH

        @pl.when(pred)
        def _():
            pltpu.sync_copy(sel_s.at[pl.ds(0, ECH)], psrc.at[pl.ds(pl.multiple_of(wid * CAP + flushed, 8), ECH)])
            pltpu.sync_copy(sel_d.at[pl.ds(0, ECH)], pdst.at[pl.ds(pl.multiple_of(wid * CAP + flushed, 8), ECH)])

            def mv(v, _):
                sel_s[pl.ds(v * 16, 16)] = sel_s[pl.ds(ECH + v * 16, 16)]
                sel_d[pl.ds(v * 16, 16)] = sel_d[pl.ds(ECH + v * 16, 16)]
                return 0

            lax.fori_loop(0, ECH // 16, mv, 0)

        staged = jnp.where(pred, staged - ECH, staged)
        flushed = jnp.where(pred, flushed + ECH, flushed)
        return staged, flushed

    staged, flushed = lax.fori_loop(0, NCH, chunk, (jnp.int32(0), jnp.int32(0)))
    pltpu.sync_copy(sel_s.at[pl.ds(0, ECH)], psrc.at[pl.ds(pl.multiple_of(wid * CAP + flushed, 8), ECH)])
    pltpu.sync_copy(sel_d.at[pl.ds(0, ECH)], pdst.at[pl.ds(pl.multiple_of(wid * CAP + flushed, 8), ECH)])
    cbuf[...] = jnp.full((16,), flushed + staged, jnp.int32)
    pltpu.sync_copy(cbuf, cnts.at[pl.ds(wid * 16, 16)])


@jax.jit
def _s0(src, dst):
    f = pl.kernel(
        _s0_body,
        mesh=_mesh,
        compiler_params=pltpu.CompilerParams(needs_layout_passes=False),
        out_type=(
            jax.ShapeDtypeStruct((32 * CAP,), jnp.int32),
            jax.ShapeDtypeStruct((32 * CAP,), jnp.int32),
            jax.ShapeDtypeStruct((32 * 16,), jnp.int32),
        ),
        scratch_types=[
            pltpu.VMEM((ECH,), jnp.int32),
            pltpu.VMEM((ECH,), jnp.int32),
            pltpu.VMEM((2 * ECH + 16,), jnp.int32),
            pltpu.VMEM((2 * ECH + 16,), jnp.int32),
            pltpu.VMEM((16,), jnp.int32),
        ],
    )
    return f(src, dst)


def _s1_body(psrc, pdst, cnts, ast_hbm, adt_hbm, mout, sout, aout,
             tab_s, tab_d, macc, sacc, sbuf_s, sbuf_d, abuf, obm, obs, cbuf):
    wid = _wid()
    d0 = wid * RANGE
    iv = lax.iota(jnp.int32, 16)
    hv = iv & 3
    i4 = iv >> 2

    pltpu.sync_copy(ast_hbm, tab_s)
    pltpu.sync_copy(adt_hbm, tab_d)
    pltpu.sync_copy(cnts.at[pl.ds(wid * 16, 16)], cbuf)
    cnt = cbuf[pl.ds(0, 16)][0]

    def init(i, _):
        macc[pl.ds(i * 16, 16)] = jnp.full((16,), -3e38, jnp.float32)
        sacc[pl.ds(i * 16, 16)] = jnp.zeros((16,), jnp.float32)
        return 0

    lax.fori_loop(0, 4 * RANGE * 4 // 16, init, 0)

    nch = (cnt + ECH - 1) // ECH

    def chunk(c, _):
        pltpu.sync_copy(psrc.at[pl.ds(pl.multiple_of(wid * CAP + c * ECH, 8), ECH)], sbuf_s)
        pltpu.sync_copy(pdst.at[pl.ds(pl.multiple_of(wid * CAP + c * ECH, 8), ECH)], sbuf_d)

        def v(vi, _):
            eids = jnp.full((16,), vi * 4, jnp.int32) + i4
            valid = eids < jnp.full((16,), cnt - c * ECH, jnp.int32)
            srcx = plsc.load_gather(sbuf_s, [eids])
            dlx = plsc.load_gather(sbuf_d, [eids])
            d0v = jnp.full((16,), d0, jnp.int32)
            a_s = plsc.load_gather(tab_s, [srcx * 4 + hv], mask=valid)
            a_d = plsc.load_gather(tab_d, [(dlx + d0v) * 4 + hv], mask=valid)
            al = a_s + a_d
            al = jnp.where(al > 0, al, 0.2 * al)
            abuf[pl.ds(vi * 16, 16)] = al
            iacc = i4 * (RANGE * 4) + dlx * 4 + hv
            mo = plsc.load_gather(macc, [iacc], mask=valid)
            so = plsc.load_gather(sacc, [iacc], mask=valid)
            mn = jnp.maximum(mo, al)
            sn = so * jnp.exp(mo - mn) + jnp.exp(al - mn)
            plsc.store_scatter(macc, [iacc], mn, mask=valid)
            plsc.store_scatter(sacc, [iacc], sn, mask=valid)
            return 0

        lax.fori_loop(0, ECH // 4, v, 0)
        pltpu.sync_copy(abuf, aout.at[pl.ds(pl.multiple_of((wid * CAP + c * ECH) * 4, 8), ECH * 4)])
        return 0

    lax.fori_loop(0, nch, chunk, 0)

    def comb(j, _):
        off = pl.ds(j * 16, 16)
        m0 = macc[pl.ds(0 * RANGE * 4 + j * 16, 16)]
        m1 = macc[pl.ds(1 * RANGE * 4 + j * 16, 16)]
        m2 = macc[pl.ds(2 * RANGE * 4 + j * 16, 16)]
        m3 = macc[pl.ds(3 * RANGE * 4 + j * 16, 16)]
        s0 = sacc[pl.ds(0 * RANGE * 4 + j * 16, 16)]
        s1 = sacc[pl.ds(1 * RANGE * 4 + j * 16, 16)]
        s2 = sacc[pl.ds(2 * RANGE * 4 + j * 16, 16)]
        s3 = sacc[pl.ds(3 * RANGE * 4 + j * 16, 16)]
        M = jnp.maximum(jnp.maximum(m0, m1), jnp.maximum(m2, m3))
        S = (s0 * jnp.exp(m0 - M) + s1 * jnp.exp(m1 - M)
             + s2 * jnp.exp(m2 - M) + s3 * jnp.exp(m3 - M))
        obm[off] = M
        obs[off] = S
        return 0

    lax.fori_loop(0, RANGE * 4 // 16, comb, 0)
    pltpu.sync_copy(obm, mout.at[pl.ds(wid * RANGE * 4, RANGE * 4)])
    pltpu.sync_copy(obs, sout.at[pl.ds(wid * RANGE * 4, RANGE * 4)])


@jax.jit
def _s1(psrc, pdst, cnts, ast, adt):
    f = pl.kernel(
        _s1_body,
        mesh=_mesh,
        compiler_params=pltpu.CompilerParams(needs_layout_passes=False),
        out_type=(
            jax.ShapeDtypeStruct((32 * RANGE * 4,), jnp.float32),
            jax.ShapeDtypeStruct((32 * RANGE * 4,), jnp.float32),
            jax.ShapeDtypeStruct((32 * CAP * 4,), jnp.float32),
        ),
        scratch_types=[
            pltpu.VMEM((NP * 4,), jnp.float32),
            pltpu.VMEM((NP * 4,), jnp.float32),
            pltpu.VMEM((4 * RANGE * 4,), jnp.float32),
            pltpu.VMEM((4 * RANGE * 4,), jnp.float32),
            pltpu.VMEM((ECH,), jnp.int32),
            pltpu.VMEM((ECH,), jnp.int32),
            pltpu.VMEM((ECH * 4,), jnp.float32),
            pltpu.VMEM((RANGE * 4,), jnp.float32),
            pltpu.VMEM((RANGE * 4,), jnp.float32),
            pltpu.VMEM((16,), jnp.int32),
        ],
    )
    return f(psrc, pdst, cnts, ast, adt)


def _s2_body(psrc, pdst, cnts, aval, mrow, srow, h_hbm, ohbm,
             acc, sbuf_s, sbuf_d, abuf, mb, sb, rows, rows2, cbuf, sem, sem2):
    wid = _wid()
    iv = lax.iota(jnp.int32, 16)
    hv = iv & 3
    i4 = iv >> 2

    pltpu.sync_copy(cnts.at[pl.ds(wid * 16, 16)], cbuf)
    cnt = cbuf[pl.ds(0, 16)][0]
    pltpu.sync_copy(mrow.at[pl.ds(wid * RANGE * 4, RANGE * 4)], mb.at[pl.ds(0, RANGE * 4)])
    pltpu.sync_copy(srow.at[pl.ds(wid * RANGE * 4, RANGE * 4)], sb.at[pl.ds(0, RANGE * 4)])

    def init(i, _):
        acc[pl.ds(i * 16, 16)] = jnp.zeros((16,), jnp.float32)
        return 0

    lax.fori_loop(0, ACCW // 16, init, 0)

    nch = (cnt + ECH - 1) // ECH

    def chunk(c, _):
        pltpu.sync_copy(psrc.at[pl.ds(pl.multiple_of(wid * CAP + c * ECH, 8), ECH)], sbuf_s)
        pltpu.sync_copy(pdst.at[pl.ds(pl.multiple_of(wid * CAP + c * ECH, 8), ECH)], sbuf_d.at[pl.ds(0, ECH)])
        pltpu.sync_copy(aval.at[pl.ds(pl.multiple_of((wid * CAP + c * ECH) * 4, 8), ECH * 4)], abuf.at[pl.ds(0, ECH * 4)])
        rem = jnp.minimum(cnt - c * ECH, ECH)

        def pre(vi, _):
            off = pl.ds(vi * 16, 16)
            sv = sbuf_s[off]
            dv = sbuf_d[off]
            valid = (jnp.full((16,), vi * 16, jnp.int32) + iv) < jnp.full((16,), rem, jnp.int32)
            sbuf_s[off] = jnp.where(valid, sv, 0)
            sbuf_d[off] = jnp.where(valid, dv, RANGE)
            return 0

        lax.fori_loop(0, ECH // 16, pre, 0)

        def pre2(vi, _):
            off = pl.ds(vi * 16, 16)
            eids = jnp.full((16,), vi * 4, jnp.int32) + i4
            dlx = plsc.load_gather(sbuf_d, [eids])
            alv = abuf[off]
            mg = plsc.load_gather(mb, [dlx * 4 + hv])
            sg = plsc.load_gather(sb, [dlx * 4 + hv])
            abuf[off] = jnp.exp(alv - mg) / (sg + 1e-16)
            return 0

        lax.fori_loop(0, ECH // 4, pre2, 0)

        nb = (rem + 15) // 16

        def process(b, rbuf):
            dvec = sbuf_d[pl.ds(b * 16, 16)]
            avecs = [abuf[pl.ds(b * 64 + j * 16, 16)] for j in range(4)]
            for i in range(16):
                dl = dvec[i]
                base = dl * 256
                av = avecs[i // 4]
                for f in range(16):
                    asc = jnp.full((16,), av[(i % 4) * 4 + f // 4], jnp.float32)
                    plsc.addupdate(acc.at[pl.ds(base + f * 16, 16)],
                                   rbuf[i, pl.ds(f * 16, 16)] * asc)

        @pl.when(nb > 0)
        def _():
            pltpu.async_copy(h_hbm.at[sbuf_s.at[pl.ds(0, 16)]], rows, sem)

        def batch(b, _):
            even = (b % 2) == 0

            @pl.when(b + 1 < nb)
            def _():
                nxt = sbuf_s.at[pl.ds((b + 1) * 16, 16)]

                @pl.when(even)
                def _():
                    pltpu.async_copy(h_hbm.at[nxt], rows2, sem2)

                @pl.when(jnp.logical_not(even))
                def _():
                    pltpu.async_copy(h_hbm.at[nxt], rows, sem)

            @pl.when(even)
            def _():
                pltpu.make_async_copy(
                    h_hbm.at[sbuf_s.at[pl.ds(b * 16, 16)]], rows, sem).wait()
                process(b, rows)

            @pl.when(jnp.logical_not(even))
            def _():
                pltpu.make_async_copy(
                    h_hbm.at[sbuf_s.at[pl.ds(b * 16, 16)]], rows2, sem2).wait()
                process(b, rows2)

            return 0

        lax.fori_loop(0, nb, batch, 0)
        return 0

    lax.fori_loop(0, nch, chunk, 0)
    pltpu.sync_copy(acc.at[pl.ds(0, RANGE * 256)], ohbm.at[pl.ds(wid * RANGE * 256, RANGE * 256)])


@jax.jit
def _s2(psrc, pdst, cnts, aval, mrow, srow, h):
    f = pl.kernel(
        _s2_body,
        mesh=_mesh,
        compiler_params=pltpu.CompilerParams(needs_layout_passes=False),
        out_type=jax.ShapeDtypeStruct((32 * RANGE * 256,), jnp.float32),
        scratch_types=[
            pltpu.VMEM((ACCW,), jnp.float32),
            pltpu.VMEM((ECH,), jnp.int32),
            pltpu.VMEM((ECH + 16,), jnp.int32),
            pltpu.VMEM((ECH * 4 + 16,), jnp.float32),
            pltpu.VMEM((RANGE * 4 + 16,), jnp.float32),
            pltpu.VMEM((RANGE * 4 + 16,), jnp.float32),
            pltpu.VMEM((16, 256), jnp.float32),
            pltpu.VMEM((16, 256), jnp.float32),
            pltpu.VMEM((16,), jnp.int32),
            pltpu.SemaphoreType.DMA,
            pltpu.SemaphoreType.DMA,
        ],
    )
    return f(psrc, pdst, cnts, aval, mrow, srow, h)


def _t1_body(x_ref, w_ref, as_ref, ad_ref, h_ref, asr_ref, adr_ref):
    h = jnp.dot(x_ref[...], w_ref[...], preferred_element_type=jnp.float32)
    h_ref[...] = h
    asr_ref[...] = jnp.dot(h, as_ref[...], preferred_element_type=jnp.float32)
    adr_ref[...] = jnp.dot(h, ad_ref[...], preferred_element_type=jnp.float32)


def _t1(x, w, As, Ad):
    d = x.shape[1]
    return pl.pallas_call(
        _t1_body,
        grid=(8,),
        in_specs=[
            pl.BlockSpec((NP // 8, d), lambda i: (i, 0)),
            pl.BlockSpec((d, 256), lambda i: (0, 0)),
            pl.BlockSpec((256, 4), lambda i: (0, 0)),
            pl.BlockSpec((256, 4), lambda i: (0, 0)),
        ],
        out_specs=[
            pl.BlockSpec((NP // 8, 256), lambda i: (i, 0)),
            pl.BlockSpec((NP // 8, 4), lambda i: (i, 0)),
            pl.BlockSpec((NP // 8, 4), lambda i: (i, 0)),
        ],
        out_shape=[
            jax.ShapeDtypeStruct((NP, 256), jnp.float32),
            jax.ShapeDtypeStruct((NP, 4), jnp.float32),
            jax.ShapeDtypeStruct((NP, 4), jnp.float32),
        ],
    )(x, w, As, Ad)


def _t2_body(o_ref, b_ref, batch_ref, hn_ref, pool_ref):
    o = o_ref[...]
    mean = (o[:, 0:64] + o[:, 64:128] + o[:, 128:192] + o[:, 192:256]) * 0.25
    hn = jnp.maximum(mean + b_ref[...], 0.0)
    hn_ref[...] = hn
    bv = batch_ref[0, 0, :]
    gid = lax.broadcasted_iota(jnp.int32, (NP // 8, NUM_GRAPHS), 1).astype(jnp.float32)
    oh = (bv[:, None] == gid).astype(jnp.float32)
    pp = lax.dot_general(oh, hn, (((0,), (0,)), ((), ())),
                         preferred_element_type=jnp.float32)

    @pl.when(pl.program_id(0) == 0)
    def _():
        pool_ref[...] = jnp.zeros_like(pool_ref)

    pool_ref[...] += pp


def _t2(o, b, batch3):
    return pl.pallas_call(
        _t2_body,
        grid=(8,),
        in_specs=[
            pl.BlockSpec((NP // 8, 256), lambda i: (i, 0)),
            pl.BlockSpec((1, HID), lambda i: (0, 0)),
            pl.BlockSpec((1, 1, NP // 8), lambda i: (i, 0, 0)),
        ],
        out_specs=[
            pl.BlockSpec((NP // 8, HID), lambda i: (i, 0)),
            pl.BlockSpec((NUM_GRAPHS, HID), lambda i: (0, 0)),
        ],
        out_shape=[
            jax.ShapeDtypeStruct((NP, HID), jnp.float32),
            jax.ShapeDtypeStruct((NUM_GRAPHS, HID), jnp.float32),
        ],
    )(o, b, batch3)


def _t3_body(p0_ref, p1_ref, p2_ref, pw_ref, pb_ref, out_ref):
    p0, p1, p2 = p0_ref[...], p1_ref[...], p2_ref[...]
    pw = pw_ref[...]
    pb = pb_ref[0, 0]
    s0 = jnp.dot(p0, pw, preferred_element_type=jnp.float32) + pb
    s1 = jnp.dot(p1, pw, preferred_element_type=jnp.float32) + pb
    s2 = jnp.dot(p2, pw, preferred_element_type=jnp.float32) + pb
    m = jnp.maximum(jnp.maximum(s0, s1), s2)
    e0 = jnp.exp(s0 - m)
    e1 = jnp.exp(s1 - m)
    e2 = jnp.exp(s2 - m)
    out_ref[...] = (e0 * p0 + e1 * p1 + e2 * p2) / (e0 + e1 + e2)


def _t3(p0, p1, p2, pw, pb):
    return pl.pallas_call(
        _t3_body,
        in_specs=[
            pl.BlockSpec((NUM_GRAPHS, HID), lambda: (0, 0)),
            pl.BlockSpec((NUM_GRAPHS, HID), lambda: (0, 0)),
            pl.BlockSpec((NUM_GRAPHS, HID), lambda: (0, 0)),
            pl.BlockSpec((HID, 1), lambda: (0, 0)),
            pl.BlockSpec((1, 1), lambda: (0, 0)),
        ],
        out_specs=pl.BlockSpec((NUM_GRAPHS, HID), lambda: (0, 0)),
        out_shape=jax.ShapeDtypeStruct((NUM_GRAPHS, HID), jnp.float32),
    )(p0, p1, p2, pw, pb)


def _attmat(att):
    return (att[0][:, :, None] * jnp.eye(HEADS, dtype=jnp.float32)[:, None, :]).reshape(HEADS * HID, HEADS)


def kernel(x, edge_index, batch, W0, att_src0, att_dst0, b0, W1, att_src1, att_dst1, b1, W2, att_src2, att_dst2, b2, proj_W, proj_b):
    loop = jnp.arange(N, dtype=jnp.int32)
    src = jnp.concatenate([edge_index[0], loop,
                           jnp.zeros((EP - ETOT,), jnp.int32)])
    dst = jnp.concatenate([edge_index[1], loop,
                           jnp.full((EP - ETOT,), NP - 1, jnp.int32)])
    psrc, pdst, cnts = _s0(src, dst)

    x_pad = jnp.concatenate([x, jnp.zeros((NP - N, IN_DIM), jnp.float32)])
    batch3 = jnp.concatenate([batch, jnp.full((NP - N,), NUM_GRAPHS, jnp.int32)]
                             ).astype(jnp.float32).reshape(8, 1, NP // 8)

    params = [(W0, att_src0, att_dst0, b0), (W1, att_src1, att_dst1, b1),
              (W2, att_src2, att_dst2, b2)]
    h = x_pad
    pooled = []
    for (W, a_s, a_d, b) in params:
        h256, asr, adr = _t1(h, W, _attmat(a_s), _attmat(a_d))
        m32, s32, aval = _s1(psrc, pdst, cnts, asr.reshape(-1), adr.reshape(-1))
        o32 = _s2(psrc, pdst, cnts, aval, m32, s32, h256)
        h, pool = _t2(o32.reshape(NP, 256), b.reshape(1, HID), batch3)
        pooled.append(pool)

    return _t3(pooled[0], pooled[1], pooled[2], proj_W, proj_b.reshape(1, 1))


# speedup vs baseline: 18.4811x; 1.0258x over previous
import functools

import jax
import jax.numpy as jnp
from jax import lax
from jax.experimental import pallas as pl
from jax.experimental.pallas import tpu as pltpu
from jax.experimental.pallas import tpu_sc as plsc

N = 10000
E = 320000
IN_DIM = 128
HID = 64
HEADS = 4
NUM_GRAPHS = 128

NP = 10240
RANGE = 320
ETOT = E + N
ECH = 2048
BCH = 8192
NCH = (ETOT + BCH - 1) // BCH
EP = NCH * BCH
CAP = EP + BCH
ACCW = (RANGE + 1) * 256

_mesh = plsc.VectorSubcoreMesh(core_axis_name="c", subcore_axis_name="s")


def _wid():
    return lax.axis_index("s") * 2 + lax.axis_index("c")


def _s0_body(src_hbm, dst_hbm, psrc, pdst, cnts, sbuf_s, sbuf_d, sel_s, sel_d, cbuf):
    wid = _wid()
    d0 = wid * RANGE

    def chunk(c, carry):
        staged, flushed = carry
        pltpu.sync_copy(src_hbm.at[pl.ds(c * BCH, BCH)], sbuf_s)
        pltpu.sync_copy(dst_hbm.at[pl.ds(c * BCH, BCH)], sbuf_d)

        def vloop(v, st):
            sv = sbuf_s[pl.ds(v * 16, 16)]
            dv = sbuf_d[pl.ds(v * 16, 16)]
            d0v = jnp.full((16,), d0, jnp.int32)
            m = (dv >= d0v) & (dv < d0v + RANGE)
            mi = jnp.where(m, jnp.full((16,), 1, jnp.int32), jnp.full((16,), 0, jnp.int32))
            cs = plsc.cumsum(mi)
            pos = jnp.full((16,), st, jnp.int32) + cs - mi
            plsc.store_scatter(sel_s, [pos], sv, mask=m)
            plsc.store_scatter(sel_d, [pos], dv - d0v, mask=m)
            return st + jnp.max(plsc.all_reduce_population_count(m))

        staged = lax.fori_loop(0, BCH // 16, vloop, staged)
        pred = staged >= BCH

        @pl.when(pred)
        def _():
            pltpu.sync_copy(sel_s.at[pl.ds(0, BCH)], psrc.at[pl.ds(pl.multiple_of(wid * CAP + flushed, 8), BCH)])
            pltpu.sync_copy(sel_d.at[pl.ds(0, BCH)], pdst.at[pl.ds(pl.multiple_of(wid * CAP + flushed, 8), BCH)])

            def mv(v, _):
                sel_s[pl.ds(v * 16, 16)] = sel_s[pl.ds(BCH + v * 16, 16)]
                sel_d[pl.ds(v * 16, 16)] = sel_d[pl.ds(BCH + v * 16, 16)]
                return 0

            lax.fori_loop(0, BCH // 16, mv, 0)

        staged = jnp.where(pred, staged - BCH, staged)
        flushed = jnp.where(pred, flushed + BCH, flushed)
        return staged, flushed

    staged, flushed = lax.fori_loop(0, NCH, chunk, (jnp.int32(0), jnp.int32(0)))
    pltpu.sync_copy(sel_s.at[pl.ds(0, BCH)], psrc.at[pl.ds(pl.multiple_of(wid * CAP + flushed, 8), BCH)])
    pltpu.sync_copy(sel_d.at[pl.ds(0, BCH)], pdst.at[pl.ds(pl.multiple_of(wid * CAP + flushed, 8), BCH)])
    cbuf[...] = jnp.full((16,), flushed + staged, jnp.int32)
    pltpu.sync_copy(cbuf, cnts.at[pl.ds(wid * 16, 16)])


@jax.jit
def _s0(src, dst):
    f = pl.kernel(
        _s0_body,
        mesh=_mesh,
        compiler_params=pltpu.CompilerParams(needs_layout_passes=False),
        out_type=(
            jax.ShapeDtypeStruct((32 * CAP,), jnp.int32),
            jax.ShapeDtypeStruct((32 * CAP,), jnp.int32),
            jax.ShapeDtypeStruct((32 * 16,), jnp.int32),
        ),
        scratch_types=[
            pltpu.VMEM((BCH,), jnp.int32),
            pltpu.VMEM((BCH,), jnp.int32),
            pltpu.VMEM((2 * BCH + 16,), jnp.int32),
            pltpu.VMEM((2 * BCH + 16,), jnp.int32),
            pltpu.VMEM((16,), jnp.int32),
        ],
    )
    return f(src, dst)


def _s1_body(psrc, pdst, cnts, ast_hbm, adt_hbm, mout, sout, aout,
             tab_s, tab_d, macc, sacc, sbuf_s, sbuf_d, abuf, obm, obs, cbuf):
    wid = _wid()
    d0 = wid * RANGE
    iv = lax.iota(jnp.int32, 16)
    hv = iv & 3
    i4 = iv >> 2

    pltpu.sync_copy(ast_hbm, tab_s)
    pltpu.sync_copy(adt_hbm, tab_d)
    pltpu.sync_copy(cnts.at[pl.ds(wid * 16, 16)], cbuf)
    cnt = cbuf[pl.ds(0, 16)][0]

    def init(i, _):
        macc[pl.ds(i * 16, 16)] = jnp.full((16,), -3e38, jnp.float32)
        sacc[pl.ds(i * 16, 16)] = jnp.zeros((16,), jnp.float32)
        return 0

    lax.fori_loop(0, 4 * RANGE * 4 // 16, init, 0)

    nch = (cnt + ECH - 1) // ECH

    def chunk(c, _):
        pltpu.sync_copy(psrc.at[pl.ds(pl.multiple_of(wid * CAP + c * ECH, 8), ECH)], sbuf_s)
        pltpu.sync_copy(pdst.at[pl.ds(pl.multiple_of(wid * CAP + c * ECH, 8), ECH)], sbuf_d)

        def v(vi, _):
            eids = jnp.full((16,), vi * 4, jnp.int32) + i4
            valid = eids < jnp.full((16,), cnt - c * ECH, jnp.int32)
            srcx = plsc.load_gather(sbuf_s, [eids])
            dlx = plsc.load_gather(sbuf_d, [eids])
            d0v = jnp.full((16,), d0, jnp.int32)
            a_s = plsc.load_gather(tab_s, [srcx * 4 + hv], mask=valid)
            a_d = plsc.load_gather(tab_d, [(dlx + d0v) * 4 + hv], mask=valid)
            al = a_s + a_d
            al = jnp.where(al > 0, al, 0.2 * al)
            abuf[pl.ds(vi * 16, 16)] = al
            iacc = i4 * (RANGE * 4) + dlx * 4 + hv
            mo = plsc.load_gather(macc, [iacc], mask=valid)
            so = plsc.load_gather(sacc, [iacc], mask=valid)
            mn = jnp.maximum(mo, al)
            sn = so * jnp.exp(mo - mn) + jnp.exp(al - mn)
            plsc.store_scatter(macc, [iacc], mn, mask=valid)
            plsc.store_scatter(sacc, [iacc], sn, mask=valid)
            return 0

        lax.fori_loop(0, ECH // 4, v, 0)
        pltpu.sync_copy(abuf, aout.at[pl.ds(pl.multiple_of((wid * CAP + c * ECH) * 4, 8), ECH * 4)])
        return 0

    lax.fori_loop(0, nch, chunk, 0)

    def comb(j, _):
        off = pl.ds(j * 16, 16)
        m0 = macc[pl.ds(0 * RANGE * 4 + j * 16, 16)]
        m1 = macc[pl.ds(1 * RANGE * 4 + j * 16, 16)]
        m2 = macc[pl.ds(2 * RANGE * 4 + j * 16, 16)]
        m3 = macc[pl.ds(3 * RANGE * 4 + j * 16, 16)]
        s0 = sacc[pl.ds(0 * RANGE * 4 + j * 16, 16)]
        s1 = sacc[pl.ds(1 * RANGE * 4 + j * 16, 16)]
        s2 = sacc[pl.ds(2 * RANGE * 4 + j * 16, 16)]
        s3 = sacc[pl.ds(3 * RANGE * 4 + j * 16, 16)]
        M = jnp.maximum(jnp.maximum(m0, m1), jnp.maximum(m2, m3))
        S = (s0 * jnp.exp(m0 - M) + s1 * jnp.exp(m1 - M)
             + s2 * jnp.exp(m2 - M) + s3 * jnp.exp(m3 - M))
        obm[off] = M
        obs[off] = S
        return 0

    lax.fori_loop(0, RANGE * 4 // 16, comb, 0)
    pltpu.sync_copy(obm, mout.at[pl.ds(wid * RANGE * 4, RANGE * 4)])
    pltpu.sync_copy(obs, sout.at[pl.ds(wid * RANGE * 4, RANGE * 4)])


@jax.jit
def _s1(psrc, pdst, cnts, ast, adt):
    f = pl.kernel(
        _s1_body,
        mesh=_mesh,
        compiler_params=pltpu.CompilerParams(needs_layout_passes=False),
        out_type=(
            jax.ShapeDtypeStruct((32 * RANGE * 4,), jnp.float32),
            jax.ShapeDtypeStruct((32 * RANGE * 4,), jnp.float32),
            jax.ShapeDtypeStruct((32 * CAP * 4,), jnp.float32),
        ),
        scratch_types=[
            pltpu.VMEM((NP * 4,), jnp.float32),
            pltpu.VMEM((NP * 4,), jnp.float32),
            pltpu.VMEM((4 * RANGE * 4,), jnp.float32),
            pltpu.VMEM((4 * RANGE * 4,), jnp.float32),
            pltpu.VMEM((ECH,), jnp.int32),
            pltpu.VMEM((ECH,), jnp.int32),
            pltpu.VMEM((ECH * 4,), jnp.float32),
            pltpu.VMEM((RANGE * 4,), jnp.float32),
            pltpu.VMEM((RANGE * 4,), jnp.float32),
            pltpu.VMEM((16,), jnp.int32),
        ],
    )
    return f(psrc, pdst, cnts, ast, adt)


def _s2_body(psrc, pdst, cnts, aval, mrow, srow, h_hbm, ohbm,
             acc, sbuf_s, sbuf_d, abuf, mb, sb, rows, rows2, cbuf, sem, sem2):
    wid = _wid()
    iv = lax.iota(jnp.int32, 16)
    hv = iv & 3
    i4 = iv >> 2

    pltpu.sync_copy(cnts.at[pl.ds(wid * 16, 16)], cbuf)
    cnt = cbuf[pl.ds(0, 16)][0]
    pltpu.sync_copy(mrow.at[pl.ds(wid * RANGE * 4, RANGE * 4)], mb.at[pl.ds(0, RANGE * 4)])
    pltpu.sync_copy(srow.at[pl.ds(wid * RANGE * 4, RANGE * 4)], sb.at[pl.ds(0, RANGE * 4)])

    def init(i, _):
        acc[pl.ds(i * 16, 16)] = jnp.zeros((16,), jnp.float32)
        return 0

    lax.fori_loop(0, ACCW // 16, init, 0)

    nch = (cnt + ECH - 1) // ECH

    def chunk(c, _):
        pltpu.sync_copy(psrc.at[pl.ds(pl.multiple_of(wid * CAP + c * ECH, 8), ECH)], sbuf_s)
        pltpu.sync_copy(pdst.at[pl.ds(pl.multiple_of(wid * CAP + c * ECH, 8), ECH)], sbuf_d.at[pl.ds(0, ECH)])
        pltpu.sync_copy(aval.at[pl.ds(pl.multiple_of((wid * CAP + c * ECH) * 4, 8), ECH * 4)], abuf.at[pl.ds(0, ECH * 4)])
        rem = jnp.minimum(cnt - c * ECH, ECH)

        def pre(vi, _):
            off = pl.ds(vi * 16, 16)
            sv = sbuf_s[off]
            dv = sbuf_d[off]
            valid = (jnp.full((16,), vi * 16, jnp.int32) + iv) < jnp.full((16,), rem, jnp.int32)
            sbuf_s[off] = jnp.where(valid, sv, 0)
            sbuf_d[off] = jnp.where(valid, dv, RANGE)
            return 0

        lax.fori_loop(0, ECH // 16, pre, 0)

        def pre2(vi, _):
            off = pl.ds(vi * 16, 16)
            eids = jnp.full((16,), vi * 4, jnp.int32) + i4
            dlx = plsc.load_gather(sbuf_d, [eids])
            alv = abuf[off]
            mg = plsc.load_gather(mb, [dlx * 4 + hv])
            sg = plsc.load_gather(sb, [dlx * 4 + hv])
            abuf[off] = jnp.exp(alv - mg) / (sg + 1e-16)
            return 0

        lax.fori_loop(0, ECH // 4, pre2, 0)

        nb = (rem + 15) // 16

        def process(b, rbuf):
            dvec = sbuf_d[pl.ds(b * 16, 16)]
            avecs = [abuf[pl.ds(b * 64 + j * 16, 16)] for j in range(4)]
            for i in range(16):
                dl = dvec[i]
                base = dl * 256
                av = avecs[i // 4]
                ah = [jnp.full((16,), av[(i % 4) * 4 + h], jnp.float32)
                      for h in range(4)]
                for f in range(16):
                    plsc.addupdate(acc.at[pl.ds(base + f * 16, 16)],
                                   rbuf[i, pl.ds(f * 16, 16)] * ah[f // 4])

        @pl.when(nb > 0)
        def _():
            pltpu.async_copy(h_hbm.at[sbuf_s.at[pl.ds(0, 16)]], rows, sem)

        def batch(b, _):
            even = (b % 2) == 0

            @pl.when(b + 1 < nb)
            def _():
                nxt = sbuf_s.at[pl.ds((b + 1) * 16, 16)]

                @pl.when(even)
                def _():
                    pltpu.async_copy(h_hbm.at[nxt], rows2, sem2)

                @pl.when(jnp.logical_not(even))
                def _():
                    pltpu.async_copy(h_hbm.at[nxt], rows, sem)

            @pl.when(even)
            def _():
                pltpu.make_async_copy(
                    h_hbm.at[sbuf_s.at[pl.ds(b * 16, 16)]], rows, sem).wait()
                process(b, rows)

            @pl.when(jnp.logical_not(even))
            def _():
                pltpu.make_async_copy(
                    h_hbm.at[sbuf_s.at[pl.ds(b * 16, 16)]], rows2, sem2).wait()
                process(b, rows2)

            return 0

        lax.fori_loop(0, nb, batch, 0)
        return 0

    lax.fori_loop(0, nch, chunk, 0)
    pltpu.sync_copy(acc.at[pl.ds(0, RANGE * 256)], ohbm.at[pl.ds(wid * RANGE * 256, RANGE * 256)])


@jax.jit
def _s2(psrc, pdst, cnts, aval, mrow, srow, h):
    f = pl.kernel(
        _s2_body,
        mesh=_mesh,
        compiler_params=pltpu.CompilerParams(needs_layout_passes=False),
        out_type=jax.ShapeDtypeStruct((32 * RANGE * 256,), jnp.float32),
        scratch_types=[
            pltpu.VMEM((ACCW,), jnp.float32),
            pltpu.VMEM((ECH,), jnp.int32),
            pltpu.VMEM((ECH + 16,), jnp.int32),
            pltpu.VMEM((ECH * 4 + 16,), jnp.float32),
            pltpu.VMEM((RANGE * 4 + 16,), jnp.float32),
            pltpu.VMEM((RANGE * 4 + 16,), jnp.float32),
            pltpu.VMEM((16, 256), jnp.float32),
            pltpu.VMEM((16, 256), jnp.float32),
            pltpu.VMEM((16,), jnp.int32),
            pltpu.SemaphoreType.DMA,
            pltpu.SemaphoreType.DMA,
        ],
    )
    return f(psrc, pdst, cnts, aval, mrow, srow, h)


def _t1_body(x_ref, w_ref, as_ref, ad_ref, h_ref, asr_ref, adr_ref):
    h = jnp.dot(x_ref[...], w_ref[...], preferred_element_type=jnp.float32)
    h_ref[...] = h
    asr_ref[...] = jnp.dot(h, as_ref[...], preferred_element_type=jnp.float32)
    adr_ref[...] = jnp.dot(h, ad_ref[...], preferred_element_type=jnp.float32)


def _t1(x, w, As, Ad):
    d = x.shape[1]
    return pl.pallas_call(
        _t1_body,
        grid=(8,),
        in_specs=[
            pl.BlockSpec((NP // 8, d), lambda i: (i, 0)),
            pl.BlockSpec((d, 256), lambda i: (0, 0)),
            pl.BlockSpec((256, 4), lambda i: (0, 0)),
            pl.BlockSpec((256, 4), lambda i: (0, 0)),
        ],
        out_specs=[
            pl.BlockSpec((NP // 8, 256), lambda i: (i, 0)),
            pl.BlockSpec((NP // 8, 4), lambda i: (i, 0)),
            pl.BlockSpec((NP // 8, 4), lambda i: (i, 0)),
        ],
        out_shape=[
            jax.ShapeDtypeStruct((NP, 256), jnp.float32),
            jax.ShapeDtypeStruct((NP, 4), jnp.float32),
            jax.ShapeDtypeStruct((NP, 4), jnp.float32),
        ],
    )(x, w, As, Ad)


def _t2_body(o_ref, b_ref, batch_ref, hn_ref, pool_ref):
    o = o_ref[...]
    mean = (o[:, 0:64] + o[:, 64:128] + o[:, 128:192] + o[:, 192:256]) * 0.25
    hn = jnp.maximum(mean + b_ref[...], 0.0)
    hn_ref[...] = hn
    bv = batch_ref[0, 0, :]
    gid = lax.broadcasted_iota(jnp.int32, (NP // 8, NUM_GRAPHS), 1).astype(jnp.float32)
    oh = (bv[:, None] == gid).astype(jnp.float32)
    pp = lax.dot_general(oh, hn, (((0,), (0,)), ((), ())),
                         preferred_element_type=jnp.float32)

    @pl.when(pl.program_id(0) == 0)
    def _():
        pool_ref[...] = jnp.zeros_like(pool_ref)

    pool_ref[...] += pp


def _t2(o, b, batch3):
    return pl.pallas_call(
        _t2_body,
        grid=(8,),
        in_specs=[
            pl.BlockSpec((NP // 8, 256), lambda i: (i, 0)),
            pl.BlockSpec((1, HID), lambda i: (0, 0)),
            pl.BlockSpec((1, 1, NP // 8), lambda i: (i, 0, 0)),
        ],
        out_specs=[
            pl.BlockSpec((NP // 8, HID), lambda i: (i, 0)),
            pl.BlockSpec((NUM_GRAPHS, HID), lambda i: (0, 0)),
        ],
        out_shape=[
            jax.ShapeDtypeStruct((NP, HID), jnp.float32),
            jax.ShapeDtypeStruct((NUM_GRAPHS, HID), jnp.float32),
        ],
    )(o, b, batch3)


def _t3_body(p0_ref, p1_ref, p2_ref, pw_ref, pb_ref, out_ref):
    p0, p1, p2 = p0_ref[...], p1_ref[...], p2_ref[...]
    pw = pw_ref[...]
    pb = pb_ref[0, 0]
    s0 = jnp.dot(p0, pw, preferred_element_type=jnp.float32) + pb
    s1 = jnp.dot(p1, pw, preferred_element_type=jnp.float32) + pb
    s2 = jnp.dot(p2, pw, preferred_element_type=jnp.float32) + pb
    m = jnp.maximum(jnp.maximum(s0, s1), s2)
    e0 = jnp.exp(s0 - m)
    e1 = jnp.exp(s1 - m)
    e2 = jnp.exp(s2 - m)
    out_ref[...] = (e0 * p0 + e1 * p1 + e2 * p2) / (e0 + e1 + e2)


def _t3(p0, p1, p2, pw, pb):
    return pl.pallas_call(
        _t3_body,
        in_specs=[
            pl.BlockSpec((NUM_GRAPHS, HID), lambda: (0, 0)),
            pl.BlockSpec((NUM_GRAPHS, HID), lambda: (0, 0)),
            pl.BlockSpec((NUM_GRAPHS, HID), lambda: (0, 0)),
            pl.BlockSpec((HID, 1), lambda: (0, 0)),
            pl.BlockSpec((1, 1), lambda: (0, 0)),
        ],
        out_specs=pl.BlockSpec((NUM_GRAPHS, HID), lambda: (0, 0)),
        out_shape=jax.ShapeDtypeStruct((NUM_GRAPHS, HID), jnp.float32),
    )(p0, p1, p2, pw, pb)


def _attmat(att):
    return (att[0][:, :, None] * jnp.eye(HEADS, dtype=jnp.float32)[:, None, :]).reshape(HEADS * HID, HEADS)


def kernel(x, edge_index, batch, W0, att_src0, att_dst0, b0, W1, att_src1, att_dst1, b1, W2, att_src2, att_dst2, b2, proj_W, proj_b):
    loop = jnp.arange(N, dtype=jnp.int32)
    src = jnp.concatenate([edge_index[0], loop,
                           jnp.zeros((EP - ETOT,), jnp.int32)])
    dst = jnp.concatenate([edge_index[1], loop,
                           jnp.full((EP - ETOT,), NP - 1, jnp.int32)])
    psrc, pdst, cnts = _s0(src, dst)

    x_pad = jnp.concatenate([x, jnp.zeros((NP - N, IN_DIM), jnp.float32)])
    batch3 = jnp.concatenate([batch, jnp.full((NP - N,), NUM_GRAPHS, jnp.int32)]
                             ).astype(jnp.float32).reshape(8, 1, NP // 8)

    params = [(W0, att_src0, att_dst0, b0), (W1, att_src1, att_dst1, b1),
              (W2, att_src2, att_dst2, b2)]
    h = x_pad
    pooled = []
    for (W, a_s, a_d, b) in params:
        h256, asr, adr = _t1(h, W, _attmat(a_s), _attmat(a_d))
        m32, s32, aval = _s1(psrc, pdst, cnts, asr.reshape(-1), adr.reshape(-1))
        o32 = _s2(psrc, pdst, cnts, aval, m32, s32, h256)
        h, pool = _t2(o32.reshape(NP, 256), b.reshape(1, HID), batch3)
        pooled.append(pool)

    return _t3(pooled[0], pooled[1], pooled[2], proj_W, proj_b.reshape(1, 1))
